# baseline (speedup 1.0000x reference)
import functools

import jax
import jax.numpy as jnp
from jax import lax
from jax.experimental import pallas as pl
from jax.experimental.pallas import tpu as pltpu
from jax.experimental.pallas import tpu_sc as plsc

B = 32
NLANES = 16
K = 3
CNB = 32
CW = 4
CQN = CNB * 128

_NEG = -3.0e38
_BIG = 1 << 30


def _insert_top3(state, x, ni):
    v1, i1, v2, i2, v3, i3 = state
    gt1 = x > v1
    gt2 = x > v2
    gt3 = x > v3
    nv1 = jnp.where(gt1, x, v1)
    ni1 = jnp.where(gt1, ni, i1)
    nv2 = jnp.where(gt1, v1, jnp.where(gt2, x, v2))
    ni2 = jnp.where(gt1, i1, jnp.where(gt2, ni, i2))
    nv3 = jnp.where(gt2, v2, jnp.where(gt3, x, v3))
    ni3 = jnp.where(gt2, i2, jnp.where(gt3, ni, i3))
    return (nv1, ni1, nv2, ni2, nv3, ni3)


@functools.partial(
    pl.kernel,
    out_type=jax.ShapeDtypeStruct((B, NLANES), jnp.int32),
    mesh=plsc.VectorSubcoreMesh(core_axis_name="c", subcore_axis_name="s"),
    scratch_types=[
        pltpu.VMEM((CNB, CW, 128), jnp.float32),
        pltpu.VMEM((CNB, CW, 128), jnp.float32),
        pltpu.VMEM((NLANES,), jnp.int32),
        pltpu.VMEM((NLANES,), jnp.int32),
        pltpu.SemaphoreType.DMA,
        pltpu.SemaphoreType.DMA,
    ],
)
def _sc_topk_count(yp_hbm, yt_hbm, out_hbm, buf0, buf1, scr, out_v,
                   sem0, sem1):
    wid = lax.axis_index("s") * 2 + lax.axis_index("c")
    iota = lax.iota(jnp.int32, NLANES)
    shuf = [iota ^ (1 << r) for r in range(4)]
    valid = iota < CW
    zero = jnp.zeros((NLANES,), jnp.int32)
    negv = jnp.full((NLANES,), _NEG, jnp.float32)

    def perm(v, idx):
        return v.at[idx].get(mode="promise_in_bounds")

    def xmax(v):
        for idx in shuf:
            v = jnp.maximum(v, perm(v, idx))
        return v

    def xmin(v):
        for idx in shuf:
            v = jnp.minimum(v, perm(v, idx))
        return v

    def xor_(v):
        for idx in shuf:
            v = v | perm(v, idx)
        return v

    def to_scalar(vec):
        scr[...] = vec
        return scr[...][0]

    def dma(a, h, wh, cq, buf, sem):
        src = yp_hbm if a == 0 else yt_hbm
        return pltpu.make_async_copy(
            src.at[wid, h, pl.ds(cq * CNB, CNB), pl.ds(wh * CW, CW), :],
            buf, sem)

    def phase_a(buf):
        def body(i, m4):
            out = []
            for w in range(CW):
                m = m4[w]
                for u in range(2):
                    for k in range(8):
                        m = jnp.maximum(
                            m, buf[i * 2 + u, w, pl.ds(k * 16, 16)])
                out.append(m)
            return tuple(out)
        m4 = lax.fori_loop(0, CNB // 2, body, (negv,) * CW)
        ts = []
        for w in range(CW):
            m = m4[w]
            m1 = xmax(m)
            m = jnp.where(m == m1, _NEG, m)
            m2 = xmax(m)
            m = jnp.where(m == m2, _NEG, m)
            ts.append(xmax(m))
        return ts

    def phase_b(buf, ts):
        def body(i, rf4):
            rf4 = list(rf4)
            nbs = jnp.full((NLANES,), 2 * i, jnp.int32)
            for w in range(CW):
                for u in range(2):
                    nb = 2 * i + u
                    hit = buf[nb, w, pl.ds(0, 16)] >= ts[w]
                    for k in range(1, 8):
                        hit = hit | (buf[nb, w, pl.ds(k * 16, 16)] >= ts[w])
                    hv = jnp.where(hit, jnp.int32(1 << u), jnp.int32(0))
                    rf4[w] = rf4[w] | (hv << nbs)
            return tuple(rf4)
        return lax.fori_loop(0, CNB // 2, body, (zero,) * CW)

    def popcnt(v):
        v = v - ((v >> 1) & 0x55555555)
        v = (v & 0x33333333) + ((v >> 2) & 0x33333333)
        v = (v + (v >> 4)) & 0x0F0F0F0F
        return (v * 0x01010101) >> 24

    def phase_c(buf, cq, rf4, states):
        us = [xor_(rf4[w]) for w in range(CW)]
        pack = us[0]
        for w in range(CW):
            if w:
                pack = jnp.where(iota == w, us[w], pack)
            pack = jnp.where(iota == CW + w, popcnt(us[w]), pack)
        scr[...] = pack
        ctrl = scr[...]

        out = []
        for w in range(CW):
            rfu0 = ctrl[w]
            mx = ctrl[CW + w]

            def body(j, carry):
                rfu, st = carry
                b = rfu & (-rfu)
                e = jnp.where(b >= (1 << 16), 16, 0)
                b2 = b >> e
                e2 = jnp.where(b2 >= (1 << 8), 8, 0)
                b3 = b2 >> e2
                e3 = jnp.where(b3 >= (1 << 4), 4, 0)
                b4 = b3 >> e3
                e4 = jnp.where(b4 >= (1 << 2), 2, 0)
                b5 = b4 >> e4
                e5 = jnp.where(b5 >= 2, 1, 0)
                row = jnp.where(b < 0, 31, e + e2 + e3 + e4 + e5)
                nb0 = cq * CQN + row * 128
                for k in range(8):
                    x = buf[row, w, pl.ds(k * 16, 16)]
                    nvec = jnp.full((NLANES,), nb0 + k * 16, jnp.int32) + iota
                    st = _insert_top3(st, x, nvec)
                return (rfu - b, st)

            _, st = lax.fori_loop(0, mx, body, (rfu0, states[w]))
            out.append(st)
        return out

    def extract_pack(states):
        pk = [zero] * K
        for w in range(CW):
            cv1, cn1, cv2, cn2, cv3, cn3 = states[w]
            for r in range(K):
                m = xmax(cv1)
                i = xmin(jnp.where(cv1 == m, cn1, _BIG))
                pk[r] = jnp.where(iota == w, i, pk[r])
                p = (cv1 == m) & (cn1 == i)
                cv1 = jnp.where(p, cv2, cv1)
                cn1 = jnp.where(p, cn2, cn1)
                cv2 = jnp.where(p, cv3, cv2)
                cn2 = jnp.where(p, cn3, cn2)
                cv3 = jnp.where(p, _NEG, cv3)
        return tuple(pk)

    def run_chunks(a, h, wh, i):
        dma(a, h, wh, 0, buf0, sem0).wait()
        ts = phase_a(buf0)
        rf4 = phase_b(buf0, ts)
        states = phase_c(buf0, 0, rf4, [(negv, zero) * 3] * CW)
        if a == 0:
            dma(1, h, wh, 0, buf0, sem0).start()
        else:
            h1 = (i + 1) >> 1
            wh1 = (i + 1) & 1

            @pl.when(i < 15)
            def _():
                dma(0, h1, wh1, 0, buf0, sem0).start()

        dma(a, h, wh, 1, buf1, sem1).wait()
        rf4 = phase_b(buf1, ts)
        states = phase_c(buf1, 1, rf4, states)
        if a == 0:
            dma(1, h, wh, 1, buf1, sem1).start()
        else:
            h1 = (i + 1) >> 1
            wh1 = (i + 1) & 1

            @pl.when(i < 15)
            def _():
                dma(0, h1, wh1, 1, buf1, sem1).start()

        return extract_pack(states)

    def outer(i, count):
        h = i >> 1
        wh = i & 1
        pk_p = run_chunks(0, h, wh, i)
        pk_t = run_chunks(1, h, wh, i)
        for r in range(K):
            count = count + jnp.where((pk_p[r] == pk_t[r]) & valid,
                                      1, 0).astype(jnp.int32)
        return count

    dma(0, 0, 0, 0, buf0, sem0).start()
    dma(0, 0, 0, 1, buf1, sem1).start()
    count = lax.fori_loop(0, 16, outer, zero)

    out_v[...] = count
    pltpu.sync_copy(out_v, out_hbm.at[wid])


def kernel(y_pred, y_true):
    def view(a):
        a = a.transpose(0, 2, 3, 1)
        a = a.reshape(32, 8, 8, 64, 128)
        return a.transpose(0, 1, 3, 2, 4)
    counts = _sc_topk_count(view(y_pred), view(y_true))
    total = jnp.float32(B * 8192 * 64 * K)
    correct = jnp.sum(counts).astype(jnp.float32)
    return jnp.float32(1.0) - correct / total


# speedup vs baseline: 1.0410x; 1.0410x over previous
import functools

import jax
import jax.numpy as jnp
from jax import lax
from jax.experimental import pallas as pl
from jax.experimental.pallas import tpu as pltpu
from jax.experimental.pallas import tpu_sc as plsc

B = 32
NLANES = 16
K = 3
CNB = 32
CW = 4
CQN = CNB * 128

_NEG = -3.0e38
_BIG = 1 << 30


def _insert_top3(state, x, ni):
    v1, i1, v2, i2, v3, i3 = state
    gt1 = x > v1
    gt2 = x > v2
    gt3 = x > v3
    nv1 = jnp.where(gt1, x, v1)
    ni1 = jnp.where(gt1, ni, i1)
    nv2 = jnp.where(gt1, v1, jnp.where(gt2, x, v2))
    ni2 = jnp.where(gt1, i1, jnp.where(gt2, ni, i2))
    nv3 = jnp.where(gt2, v2, jnp.where(gt3, x, v3))
    ni3 = jnp.where(gt2, i2, jnp.where(gt3, ni, i3))
    return (nv1, ni1, nv2, ni2, nv3, ni3)


@functools.partial(
    pl.kernel,
    out_type=jax.ShapeDtypeStruct((B, NLANES), jnp.int32),
    mesh=plsc.VectorSubcoreMesh(core_axis_name="c", subcore_axis_name="s"),
    scratch_types=[
        pltpu.VMEM((CNB, CW, 128), jnp.float32),
        pltpu.VMEM((CNB, CW, 128), jnp.float32),
        pltpu.VMEM((NLANES,), jnp.int32),
        pltpu.VMEM((NLANES,), jnp.int32),
        pltpu.SemaphoreType.DMA,
        pltpu.SemaphoreType.DMA,
    ],
)
def _sc_topk_count(yp_hbm, yt_hbm, out_hbm, buf0, buf1, scr, out_v,
                   sem0, sem1):
    wid = lax.axis_index("s") * 2 + lax.axis_index("c")
    iota = lax.iota(jnp.int32, NLANES)
    shuf = [iota ^ (1 << r) for r in range(4)]
    valid = iota < CW
    zero = jnp.zeros((NLANES,), jnp.int32)
    negv = jnp.full((NLANES,), _NEG, jnp.float32)

    def perm(v, idx):
        return v.at[idx].get(mode="promise_in_bounds")

    def xmax(v):
        for idx in shuf:
            v = jnp.maximum(v, perm(v, idx))
        return v

    def xmin(v):
        for idx in shuf:
            v = jnp.minimum(v, perm(v, idx))
        return v

    def xor_(v):
        for idx in shuf:
            v = v | perm(v, idx)
        return v

    def dma(a, h, wh, cq, buf, sem):
        src = yp_hbm if a == 0 else yt_hbm
        return pltpu.make_async_copy(
            src.at[wid, h, pl.ds(cq * CNB, CNB), pl.ds(wh * CW, CW), :],
            buf, sem)

    def phase_a(buf):
        def body(nb, m4):
            out = []
            for w in range(CW):
                m = m4[w]
                for k in range(8):
                    m = jnp.maximum(m, buf[nb, w, pl.ds(k * 16, 16)])
                out.append(m)
            return tuple(out)
        m4 = lax.fori_loop(0, CNB, body, (negv,) * CW)
        ts = []
        for w in range(CW):
            m = m4[w]
            m1 = xmax(m)
            m = jnp.where(m == m1, _NEG, m)
            m2 = xmax(m)
            m = jnp.where(m == m2, _NEG, m)
            ts.append(xmax(m))
        return ts

    def phase_b(buf, ts):
        def body(nb, rf4):
            rf4 = list(rf4)
            nbs = jnp.full((NLANES,), nb, jnp.int32)
            for w in range(CW):
                hit = buf[nb, w, pl.ds(0, 16)] >= ts[w]
                for k in range(1, 8):
                    hit = hit | (buf[nb, w, pl.ds(k * 16, 16)] >= ts[w])
                hv = jnp.where(hit, jnp.int32(1), jnp.int32(0))
                rf4[w] = rf4[w] | (hv << nbs)
            return tuple(rf4)
        return lax.fori_loop(0, CNB, body, (zero,) * CW)

    def popcnt(v):
        v = v - ((v >> 1) & 0x55555555)
        v = (v & 0x33333333) + ((v >> 2) & 0x33333333)
        v = (v + (v >> 4)) & 0x0F0F0F0F
        return (v * 0x01010101) >> 24

    def phase_c(buf, cq, rf4, states):
        us = [xor_(rf4[w]) for w in range(CW)]
        pack = us[0]
        for w in range(CW):
            if w:
                pack = jnp.where(iota == w, us[w], pack)
            pack = jnp.where(iota == CW + w, popcnt(us[w]), pack)
        scr[...] = pack
        ctrl = scr[...]

        out = []
        for w in range(CW):
            rfu0 = ctrl[w]
            mx = ctrl[CW + w]

            def body(j, carry):
                rfu, st = carry
                b = rfu & (-rfu)
                e = jnp.where(b >= (1 << 16), 16, 0)
                b2 = b >> e
                e2 = jnp.where(b2 >= (1 << 8), 8, 0)
                b3 = b2 >> e2
                e3 = jnp.where(b3 >= (1 << 4), 4, 0)
                b4 = b3 >> e3
                e4 = jnp.where(b4 >= (1 << 2), 2, 0)
                b5 = b4 >> e4
                e5 = jnp.where(b5 >= 2, 1, 0)
                row = jnp.where(b < 0, 31, e + e2 + e3 + e4 + e5)
                nb0 = cq * CQN + row * 128
                for k in range(8):
                    x = buf[row, w, pl.ds(k * 16, 16)]
                    nvec = jnp.full((NLANES,), nb0 + k * 16, jnp.int32) + iota
                    st = _insert_top3(st, x, nvec)
                return (rfu - b, st)

            _, st = lax.fori_loop(0, mx, body, (rfu0, states[w]))
            out.append(st)
        return out

    def extract_pack(states):
        pk = [zero] * K
        for w in range(CW):
            cv1, cn1, cv2, cn2, cv3, cn3 = states[w]
            for r in range(K):
                m = xmax(cv1)
                i = xmin(jnp.where(cv1 == m, cn1, _BIG))
                pk[r] = jnp.where(iota == w, i, pk[r])
                p = (cv1 == m) & (cn1 == i)
                cv1 = jnp.where(p, cv2, cv1)
                cn1 = jnp.where(p, cn2, cn1)
                cv2 = jnp.where(p, cv3, cv2)
                cn2 = jnp.where(p, cn3, cn2)
                cv3 = jnp.where(p, _NEG, cv3)
        return tuple(pk)

    def run_chunks(a, h, wh, i):
        dma(a, h, wh, 0, buf0, sem0).wait()
        ts = phase_a(buf0)
        rf4 = phase_b(buf0, ts)
        states = phase_c(buf0, 0, rf4, [(negv, zero) * 3] * CW)
        if a == 0:
            dma(1, h, wh, 0, buf0, sem0).start()
        else:
            h1 = (i + 1) >> 1
            wh1 = (i + 1) & 1

            @pl.when(i < 15)
            def _():
                dma(0, h1, wh1, 0, buf0, sem0).start()

        dma(a, h, wh, 1, buf1, sem1).wait()
        rf4 = phase_b(buf1, ts)
        states = phase_c(buf1, 1, rf4, states)
        if a == 0:
            dma(1, h, wh, 1, buf1, sem1).start()
        else:
            h1 = (i + 1) >> 1
            wh1 = (i + 1) & 1

            @pl.when(i < 15)
            def _():
                dma(0, h1, wh1, 1, buf1, sem1).start()

        return extract_pack(states)

    def outer(i, count):
        h = i >> 1
        wh = i & 1
        pk_p = run_chunks(0, h, wh, i)
        pk_t = run_chunks(1, h, wh, i)
        for r in range(K):
            count = count + jnp.where((pk_p[r] == pk_t[r]) & valid,
                                      1, 0).astype(jnp.int32)
        return count

    dma(0, 0, 0, 0, buf0, sem0).start()
    dma(0, 0, 0, 1, buf1, sem1).start()
    count = lax.fori_loop(0, 16, outer, zero)

    out_v[...] = count
    pltpu.sync_copy(out_v, out_hbm.at[wid])


def kernel(y_pred, y_true):
    def view(a):
        a = a.transpose(0, 2, 3, 1)
        a = a.reshape(32, 8, 8, 64, 128)
        return a.transpose(0, 1, 3, 2, 4)
    counts = _sc_topk_count(view(y_pred), view(y_true))
    total = jnp.float32(B * 8192 * 64 * K)
    correct = jnp.sum(counts).astype(jnp.float32)
    return jnp.float32(1.0) - correct / total
